# baseline (speedup 1.0000x reference)
import functools

import jax
import jax.numpy as jnp
from jax import lax
from jax.experimental import pallas as pl
from jax.experimental.pallas import tpu as pltpu
from jax.experimental.pallas import tpu_sc as plsc

B = 2
N = 100000
NPAD = 102400
BEV_H = 256
BEV_W = 256
CELLS = BEV_H * BEV_W
PFN_OUT = 64
BEV_CH = 128
FTBL = 128

NTILES = 16
PTS_PER_TILE = NPAD // NTILES
ASTEPS = PTS_PER_TILE // 16
NSLOTS = 64
SLOT_CELLS = CELLS // NSLOTS
NPASSES = NSLOTS // NTILES
ENT_CAP = PTS_PER_TILE + 8 * NSLOTS
ENT_ROW = ENT_CAP + 1024
CHUNK = 1024
GRP = 128
DUMMY_PIDX = NPAD - 1


def _pfn_body(ptsT_ref, w_ref, b_ref, feat_ref, ids_ref):
    ci = pl.program_id(1)
    x = ptsT_ref[0]
    f = jax.nn.relu(
        lax.dot_general(x[0:4, :], w_ref[...], (((0,), (0,)), ((), ())),
                        preferred_element_type=jnp.float32)
        + b_ref[...][None, :]
    )
    base = ci * f.shape[0]
    colv = base + lax.broadcasted_iota(jnp.int32, (f.shape[0], 1), 0)
    feat_ref[0] = jnp.where(colv < N, f, 0.0)
    col = base + lax.broadcasted_iota(jnp.int32, (1, f.shape[0]), 1)
    valid = col < N
    ix = jnp.clip((x[0:1, :] * BEV_W).astype(jnp.int32), 0, BEV_W - 1)
    iy = jnp.clip((x[1:2, :] * BEV_H).astype(jnp.int32), 0, BEV_H - 1)
    ids = iy * BEV_W + ix
    ids_ref[0] = jnp.where(valid, ids, CELLS - 1)


def _run_pfn(ptsT, W_pfn, b_pfn):
    CHN = 4096
    nch = NPAD // CHN
    feat, ids = pl.pallas_call(
        _pfn_body,
        grid=(B, nch),
        in_specs=[
            pl.BlockSpec((1, 5, CHN), lambda b, c: (b, 0, c)),
            pl.BlockSpec((4, FTBL), lambda b, c: (0, 0)),
            pl.BlockSpec((FTBL,), lambda b, c: (0,)),
        ],
        out_specs=[
            pl.BlockSpec((1, CHN, FTBL), lambda b, c: (b, c, 0)),
            pl.BlockSpec((1, 1, CHN), lambda b, c: (b, 0, c)),
        ],
        out_shape=[
            jax.ShapeDtypeStruct((B, NPAD, FTBL), jnp.float32),
            jax.ShapeDtypeStruct((B, 1, NPAD), jnp.int32),
        ],
    )(ptsT, W_pfn, b_pfn)
    return feat, ids


def _scatter_body(ids_hbm, feat_hbm, bev_hbm,
                  ids_v, counts2, offsets2, entries_v, starts_v, cnts_v,
                  startsm, cntsm, chunk_v, idx_v, cells_v, rows_v, slab,
                  entries_s, starts_s, cnts_s, sem):
    c = lax.axis_index("c")
    sid = lax.axis_index("s")
    lane = lax.iota(jnp.int32, 16)
    zeros16 = jnp.zeros((16,), jnp.int32)
    lane0 = lane < 1

    pltpu.sync_copy(ids_hbm.at[c, 0, pl.ds(sid * PTS_PER_TILE, PTS_PER_TILE)],
                    ids_v)

    def _zero(k, _):
        counts2[pl.ds(k * 16, 16)] = zeros16
        return 0
    lax.fori_loop(0, NSLOTS, _zero, 0)

    def _count(i, _):
        cv = ids_v[pl.ds(i * 16, 16)]
        slot16 = (lax.shift_right_logical(cv, 10) * 16) + lane
        cur = plsc.load_gather(counts2, [slot16])
        plsc.store_scatter(counts2, [slot16], cur + 1)
        return 0
    lax.fori_loop(0, ASTEPS, _count, 0)

    def _scan(r, carry):
        v = counts2[pl.ds(r * 16, 16)]
        s = plsc.cumsum(v)
        tot = jnp.sum(v)
        offsets2[pl.ds(r * 16, 16)] = (s - v) + carry
        r16 = jnp.broadcast_to(r, (16,))
        plsc.store_scatter(starts_v, [r16],
                           jnp.broadcast_to(carry, (16,)), mask=lane0)
        plsc.store_scatter(cnts_v, [r16],
                           jnp.broadcast_to(tot, (16,)), mask=lane0)
        return carry + ((tot + 7) & ~7)
    lax.fori_loop(0, NSLOTS, _scan, jnp.int32(0))

    def _place(i, _):
        cv = ids_v[pl.ds(i * 16, 16)]
        slot16 = (lax.shift_right_logical(cv, 10) * 16) + lane
        o = plsc.load_gather(offsets2, [slot16])
        plsc.store_scatter(offsets2, [slot16], o + 1)
        pidx = sid * PTS_PER_TILE + i * 16 + lane
        entry = ((cv & (SLOT_CELLS - 1)) << 17) | pidx
        plsc.store_scatter(entries_v, [o], entry)
        return 0
    lax.fori_loop(0, ASTEPS, _place, 0)

    pltpu.sync_copy(
        entries_v,
        entries_s.at[pl.ds(pl.multiple_of(sid * ENT_ROW, 8), ENT_CAP)])
    pltpu.sync_copy(
        starts_v,
        starts_s.at[pl.ds(pl.multiple_of(sid * NSLOTS, 8), NSLOTS)])
    pltpu.sync_copy(
        cnts_v,
        cnts_s.at[pl.ds(pl.multiple_of(sid * NSLOTS, 8), NSLOTS)])
    plsc.subcore_barrier()

    pltpu.sync_copy(starts_s, startsm)
    pltpu.sync_copy(cnts_s, cntsm)

    fbase = c * NPAD

    for p in range(NPASSES):
        o = p * NTILES + sid
        o16 = jnp.broadcast_to(o, (16,))

        def _zslab(k, _):
            slab[pl.ds(k * 16, 16)] = jnp.zeros((16,), jnp.float32)
            return 0
        lax.fori_loop(0, SLOT_CELLS * PFN_OUT // 16, _zslab, 0)

        def _src(s, _):
            so16 = jnp.broadcast_to(s * NSLOTS, (16,)) + o16
            start = plsc.load_gather(startsm, [so16])[0]
            cnt = plsc.load_gather(cntsm, [so16])[0]

            def _chunk_cond(st):
                return st[0] < cnt

            def _chunk(st):
                done = st[0]
                pltpu.sync_copy(
                    entries_s.at[pl.ds(
                        pl.multiple_of(s * ENT_ROW + start + done, 8), CHUNK)],
                    chunk_v)
                navail = jnp.minimum(cnt - done, CHUNK)

                def _grp_cond(gt):
                    return gt[0] < navail

                def _grp(gt):
                    g0 = gt[0]
                    def _prep(j, _):
                        e = chunk_v[pl.ds(g0 + j * 16, 16)]
                        pos = g0 + j * 16 + lane
                        ok = pos < navail
                        pidx = jnp.where(ok, e & (131072 - 1), DUMMY_PIDX)
                        cell = jnp.where(
                            ok, lax.shift_right_logical(e, 17), SLOT_CELLS - 1)
                        idx_v[pl.ds(j * 16, 16)] = pidx + fbase
                        cells_v[pl.ds(j * 16, 16)] = cell
                        return 0
                    lax.fori_loop(0, GRP // 16, _prep, 0)
                    pltpu.async_copy(feat_hbm.at[idx_v], rows_v, sem).wait()

                    def _rmw(j, _):
                        cv16 = cells_v[pl.ds(j * 16, 16)]
                        for k in range(16):
                            cell = cv16[k]
                            base = cell * PFN_OUT
                            g = j * 16 + k
                            for q in range(PFN_OUT // 16):
                                cur = slab[pl.ds(base + q * 16, 16)]
                                fv = rows_v[g, pl.ds(q * 16, 16)]
                                slab[pl.ds(base + q * 16, 16)] = \
                                    jnp.maximum(cur, fv)
                        return 0
                    lax.fori_loop(0, GRP // 16, _rmw, 0)
                    return (g0 + GRP,)
                lax.while_loop(_grp_cond, _grp, (jnp.int32(0),))
                return (done + CHUNK,)
            lax.while_loop(_chunk_cond, _chunk, (jnp.int32(0),))
            return 0
        lax.fori_loop(0, NTILES, _src, 0)

        pltpu.sync_copy(
            slab,
            bev_hbm.at[pl.ds(
                pl.multiple_of(
                    c * CELLS * PFN_OUT + o * SLOT_CELLS * PFN_OUT, 8),
                SLOT_CELLS * PFN_OUT)])


def _run_scatter(ids, feat):
    mesh = plsc.VectorSubcoreMesh(core_axis_name="c", subcore_axis_name="s")
    k = functools.partial(
        pl.kernel,
        mesh=mesh,
        compiler_params=pltpu.CompilerParams(needs_layout_passes=False),
        out_type=jax.ShapeDtypeStruct((B * CELLS * PFN_OUT,), jnp.float32),
        scratch_types=[
            pltpu.VMEM((PTS_PER_TILE,), jnp.int32),
            pltpu.VMEM((NSLOTS * 16,), jnp.int32),
            pltpu.VMEM((NSLOTS * 16,), jnp.int32),
            pltpu.VMEM((ENT_CAP,), jnp.int32),
            pltpu.VMEM((NSLOTS,), jnp.int32),
            pltpu.VMEM((NSLOTS,), jnp.int32),
            pltpu.VMEM((NTILES * NSLOTS,), jnp.int32),
            pltpu.VMEM((NTILES * NSLOTS,), jnp.int32),
            pltpu.VMEM((CHUNK,), jnp.int32),
            pltpu.VMEM((GRP,), jnp.int32),
            pltpu.VMEM((GRP,), jnp.int32),
            pltpu.VMEM((GRP, FTBL), jnp.float32),
            pltpu.VMEM((SLOT_CELLS * PFN_OUT,), jnp.float32),
            pltpu.VMEM_SHARED((NTILES * ENT_ROW,), jnp.int32),
            pltpu.VMEM_SHARED((NTILES * NSLOTS,), jnp.int32),
            pltpu.VMEM_SHARED((NTILES * NSLOTS,), jnp.int32),
            pltpu.SemaphoreType.DMA,
        ],
    )(_scatter_body)
    bev = k(ids, feat.reshape(B * NPAD, FTBL))
    return bev.reshape(B, CELLS, PFN_OUT)


H2 = BEV_H // 2


def _conv1_body(par_ref, w_ref, b_ref, y1_ref):
    acc = jnp.zeros((H2 * H2, PFN_OUT), jnp.float32)
    for di in range(3):
        rp = di % 2
        rs = 1 if di == 2 else 0
        xa = par_ref[0, rp, rs:rs + H2, 0:H2, :].reshape(H2 * H2, 2 * PFN_OUT)
        acc = acc + jnp.dot(xa, w_ref[di * 2],
                            preferred_element_type=jnp.float32)
        xb = par_ref[0, rp, rs:rs + H2, 1:H2 + 1, :].reshape(
            H2 * H2, 2 * PFN_OUT)
        acc = acc + jnp.dot(xb, w_ref[di * 2 + 1],
                            preferred_element_type=jnp.float32)
    y1 = jax.nn.relu(acc + b_ref[...][None, :].astype(jnp.float32))
    y1_ref[0] = jnp.pad(y1.reshape(H2, H2, PFN_OUT),
                        ((1, 1), (1, 1), (0, 0)))


def _conv2_body(y1_ref, w_ref, b_ref, out_ref):
    acc = jnp.zeros((H2 * H2, BEV_CH), jnp.float32)
    for di in range(3):
        for dj in range(3):
            xk = y1_ref[0, di:di + H2, dj:dj + H2, :].reshape(
                H2 * H2, PFN_OUT)
            acc = acc + jnp.dot(xk, w_ref[di * 3 + dj],
                                preferred_element_type=jnp.float32)
    out_ref[0] = jax.nn.relu(acc + b_ref[...][None, :]).reshape(
        H2, H2, BEV_CH)


def _run_convs(par, w1p, b1, w2t, b2):
    y1 = pl.pallas_call(
        _conv1_body,
        grid=(B,),
        compiler_params=pltpu.CompilerParams(
            vmem_limit_bytes=100 * 1024 * 1024),
        in_specs=[
            pl.BlockSpec((1, 2, H2 + 1, H2 + 1, 2 * PFN_OUT),
                         lambda b: (b, 0, 0, 0, 0)),
            pl.BlockSpec((6, 2 * PFN_OUT, PFN_OUT), lambda b: (0, 0, 0)),
            pl.BlockSpec((PFN_OUT,), lambda b: (0,)),
        ],
        out_specs=pl.BlockSpec((1, H2 + 2, H2 + 2, PFN_OUT),
                               lambda b: (b, 0, 0, 0)),
        out_shape=jax.ShapeDtypeStruct((B, H2 + 2, H2 + 2, PFN_OUT),
                                       jnp.float32),
    )(par, w1p, b1)
    out = pl.pallas_call(
        _conv2_body,
        grid=(B,),
        compiler_params=pltpu.CompilerParams(
            vmem_limit_bytes=100 * 1024 * 1024),
        in_specs=[
            pl.BlockSpec((1, H2 + 2, H2 + 2, PFN_OUT),
                         lambda b: (b, 0, 0, 0)),
            pl.BlockSpec((9, PFN_OUT, BEV_CH), lambda b: (0, 0, 0)),
            pl.BlockSpec((BEV_CH,), lambda b: (0,)),
        ],
        out_specs=pl.BlockSpec((1, H2, H2, BEV_CH), lambda b: (b, 0, 0, 0)),
        out_shape=jax.ShapeDtypeStruct((B, H2, H2, BEV_CH), jnp.float32),
    )(y1, w2t, b2)
    return out


def kernel(points, W_pfn, b_pfn, conv1_w, conv1_b, conv2_w, conv2_b):
    pts = jnp.pad(points, ((0, 0), (0, NPAD - N), (0, 0)))
    ptsT = pts.transpose(0, 2, 1)
    w_pad = jnp.pad(W_pfn, ((0, 0), (0, FTBL - PFN_OUT)))
    b_pad = jnp.pad(b_pfn, (0, FTBL - PFN_OUT))
    feat, ids = _run_pfn(ptsT, w_pad, b_pad)

    bev = _run_scatter(ids, feat)

    x = bev.reshape(B, H2, 2, H2, 2, PFN_OUT)
    par = x.transpose(0, 2, 1, 3, 4, 5).reshape(B, 2, H2, H2, 2 * PFN_OUT)
    par = jnp.pad(par, ((0, 0), (0, 0), (0, 1), (0, 1), (0, 0)))
    par = par.astype(jnp.bfloat16)

    w1t = conv1_w.transpose(2, 3, 1, 0)
    wz = jnp.zeros((PFN_OUT, PFN_OUT), jnp.float32)
    w1p = jnp.stack([
        jnp.concatenate([w1t[0, 0], w1t[0, 1]], axis=0),
        jnp.concatenate([w1t[0, 2], wz], axis=0),
        jnp.concatenate([w1t[1, 0], w1t[1, 1]], axis=0),
        jnp.concatenate([w1t[1, 2], wz], axis=0),
        jnp.concatenate([w1t[2, 0], w1t[2, 1]], axis=0),
        jnp.concatenate([w1t[2, 2], wz], axis=0),
    ]).astype(jnp.bfloat16)
    w2t = conv2_w.transpose(2, 3, 1, 0).reshape(9, PFN_OUT, BEV_CH)
    out = _run_convs(par, w1p, conv1_b, w2t, conv2_b)
    return out.transpose(0, 3, 1, 2)


# speedup vs baseline: 2.8627x; 2.8627x over previous
import functools

import jax
import jax.numpy as jnp
from jax import lax
from jax.experimental import pallas as pl
from jax.experimental.pallas import tpu as pltpu
from jax.experimental.pallas import tpu_sc as plsc

B = 2
N = 100000
NPAD = 102400
BEV_H = 256
BEV_W = 256
CELLS = BEV_H * BEV_W
PFN_OUT = 64
BEV_CH = 128
FTBL = 128

NTILES = 16
PTS_PER_TILE = NPAD // NTILES
ASTEPS = PTS_PER_TILE // 16
NSLOTS = 64
SLOT_CELLS = CELLS // NSLOTS
NPASSES = NSLOTS // NTILES
ENT_CAP = PTS_PER_TILE + 8 * NSLOTS
ENT_ROW = ENT_CAP + 1024
CHUNK = 1024
GRP = 128
DUMMY_PIDX = NPAD - 1


def _pfn_body(ptsT_ref, w_ref, b_ref, feat_ref, ids_ref):
    ci = pl.program_id(1)
    x = ptsT_ref[0]
    f = jax.nn.relu(
        lax.dot_general(x[0:4, :], w_ref[...], (((0,), (0,)), ((), ())),
                        preferred_element_type=jnp.float32)
        + b_ref[...][None, :]
    )
    base = ci * f.shape[0]
    colv = base + lax.broadcasted_iota(jnp.int32, (f.shape[0], 1), 0)
    feat_ref[0] = jnp.where(colv < N, f, 0.0)
    col = base + lax.broadcasted_iota(jnp.int32, (1, f.shape[0]), 1)
    valid = col < N
    ix = jnp.clip((x[0:1, :] * BEV_W).astype(jnp.int32), 0, BEV_W - 1)
    iy = jnp.clip((x[1:2, :] * BEV_H).astype(jnp.int32), 0, BEV_H - 1)
    ids = iy * BEV_W + ix
    ids_ref[0] = jnp.where(valid, ids, CELLS - 1)


def _run_pfn(ptsT, W_pfn, b_pfn):
    CHN = 4096
    nch = NPAD // CHN
    feat, ids = pl.pallas_call(
        _pfn_body,
        grid=(B, nch),
        in_specs=[
            pl.BlockSpec((1, 5, CHN), lambda b, c: (b, 0, c)),
            pl.BlockSpec((4, FTBL), lambda b, c: (0, 0)),
            pl.BlockSpec((FTBL,), lambda b, c: (0,)),
        ],
        out_specs=[
            pl.BlockSpec((1, CHN, FTBL), lambda b, c: (b, c, 0)),
            pl.BlockSpec((1, 1, CHN), lambda b, c: (b, 0, c)),
        ],
        out_shape=[
            jax.ShapeDtypeStruct((B, NPAD, FTBL), jnp.float32),
            jax.ShapeDtypeStruct((B, 1, NPAD), jnp.int32),
        ],
    )(ptsT, W_pfn, b_pfn)
    return feat, ids


def _scatter_body(ids_hbm, feat_hbm, bev_hbm,
                  ids_v, counts2, offsets2, entries_v, starts_v, cnts_v,
                  startsm, cntsm, chunk_v, idx_v, cells_v, rows_v, slab,
                  entries_s, starts_s, cnts_s, sem):
    c = lax.axis_index("c")
    sid = lax.axis_index("s")
    lane = lax.iota(jnp.int32, 16)
    zeros16 = jnp.zeros((16,), jnp.int32)
    lane0 = lane < 1

    pltpu.sync_copy(ids_hbm.at[c, 0, pl.ds(sid * PTS_PER_TILE, PTS_PER_TILE)],
                    ids_v)

    def _zero(k, _):
        counts2[pl.ds(k * 16, 16)] = zeros16
        return 0
    lax.fori_loop(0, NSLOTS, _zero, 0)

    def _count(i, _):
        cv = ids_v[pl.ds(i * 16, 16)]
        slot16 = (lax.shift_right_logical(cv, 10) * 16) + lane
        cur = plsc.load_gather(counts2, [slot16])
        plsc.store_scatter(counts2, [slot16], cur + 1)
        return 0
    lax.fori_loop(0, ASTEPS, _count, 0)

    def _scan(r, carry):
        v = counts2[pl.ds(r * 16, 16)]
        s = plsc.cumsum(v)
        tot = jnp.sum(v)
        offsets2[pl.ds(r * 16, 16)] = (s - v) + carry
        r16 = jnp.broadcast_to(r, (16,))
        plsc.store_scatter(starts_v, [r16],
                           jnp.broadcast_to(carry, (16,)), mask=lane0)
        plsc.store_scatter(cnts_v, [r16],
                           jnp.broadcast_to(tot, (16,)), mask=lane0)
        return carry + ((tot + 7) & ~7)
    lax.fori_loop(0, NSLOTS, _scan, jnp.int32(0))

    def _place(i, _):
        cv = ids_v[pl.ds(i * 16, 16)]
        slot16 = (lax.shift_right_logical(cv, 10) * 16) + lane
        o = plsc.load_gather(offsets2, [slot16])
        plsc.store_scatter(offsets2, [slot16], o + 1)
        pidx = sid * PTS_PER_TILE + i * 16 + lane
        entry = ((cv & (SLOT_CELLS - 1)) << 17) | pidx
        plsc.store_scatter(entries_v, [o], entry)
        return 0
    lax.fori_loop(0, ASTEPS, _place, 0)

    pltpu.sync_copy(
        entries_v,
        entries_s.at[pl.ds(pl.multiple_of(sid * ENT_ROW, 8), ENT_CAP)])
    pltpu.sync_copy(
        starts_v,
        starts_s.at[pl.ds(pl.multiple_of(sid * NSLOTS, 8), NSLOTS)])
    pltpu.sync_copy(
        cnts_v,
        cnts_s.at[pl.ds(pl.multiple_of(sid * NSLOTS, 8), NSLOTS)])
    plsc.subcore_barrier()

    pltpu.sync_copy(starts_s, startsm)
    pltpu.sync_copy(cnts_s, cntsm)

    fbase = c * NPAD

    for p in range(NPASSES):
        o = p * NTILES + sid
        o16 = jnp.broadcast_to(o, (16,))

        def _zslab(k, _):
            for u in range(4):
                slab[pl.ds(k * 64 + u * 16, 16)] = jnp.zeros((16,),
                                                             jnp.float32)
            return 0
        lax.fori_loop(0, SLOT_CELLS * PFN_OUT // 64, _zslab, 0)

        def _src(s, _):
            so16 = jnp.broadcast_to(s * NSLOTS, (16,)) + o16
            start = plsc.load_gather(startsm, [so16])[0]
            cnt = plsc.load_gather(cntsm, [so16])[0]

            def _chunk_cond(st):
                return st[0] < cnt

            def _chunk(st):
                done = st[0]
                pltpu.sync_copy(
                    entries_s.at[pl.ds(
                        pl.multiple_of(s * ENT_ROW + start + done, 8), CHUNK)],
                    chunk_v)
                navail = jnp.minimum(cnt - done, CHUNK)

                def _grp_cond(gt):
                    return gt[0] < navail

                def _grp(gt):
                    g0 = gt[0]
                    def _prep(j, _):
                        e = chunk_v[pl.ds(g0 + j * 16, 16)]
                        pos = g0 + j * 16 + lane
                        ok = pos < navail
                        pidx = jnp.where(ok, e & (131072 - 1),
                                         N + j * 16 + lane)
                        cell = jnp.where(
                            ok, lax.shift_right_logical(e, 17), SLOT_CELLS - 1)
                        idx_v[pl.ds(j * 16, 16)] = pidx + fbase
                        cells_v[pl.ds(j * 16, 16)] = cell
                        return 0
                    lax.fori_loop(0, GRP // 16, _prep, 0)
                    pltpu.async_copy(feat_hbm.at[idx_v], rows_v, sem).wait()

                    def _rmw(j, _):
                        cv16 = cells_v[pl.ds(j * 16, 16)]
                        for k in range(16):
                            cell = cv16[k]
                            base = cell * PFN_OUT
                            g = j * 16 + k
                            nq = PFN_OUT // 16
                            curs = [slab[pl.ds(base + q * 16, 16)]
                                    for q in range(nq)]
                            fvs = [rows_v[g, pl.ds(q * 16, 16)]
                                   for q in range(nq)]
                            for q in range(nq):
                                slab[pl.ds(base + q * 16, 16)] = \
                                    jnp.maximum(curs[q], fvs[q])
                        return 0
                    lax.fori_loop(0, GRP // 16, _rmw, 0)
                    return (g0 + GRP,)
                lax.while_loop(_grp_cond, _grp, (jnp.int32(0),))
                return (done + CHUNK,)
            lax.while_loop(_chunk_cond, _chunk, (jnp.int32(0),))
            return 0
        lax.fori_loop(0, NTILES, _src, 0)

        pltpu.sync_copy(
            slab,
            bev_hbm.at[pl.ds(
                pl.multiple_of(
                    c * CELLS * PFN_OUT + o * SLOT_CELLS * PFN_OUT, 8),
                SLOT_CELLS * PFN_OUT)])


def _run_scatter(ids, feat):
    mesh = plsc.VectorSubcoreMesh(core_axis_name="c", subcore_axis_name="s")
    k = functools.partial(
        pl.kernel,
        mesh=mesh,
        compiler_params=pltpu.CompilerParams(needs_layout_passes=False),
        out_type=jax.ShapeDtypeStruct((B * CELLS * PFN_OUT,), jnp.float32),
        scratch_types=[
            pltpu.VMEM((PTS_PER_TILE,), jnp.int32),
            pltpu.VMEM((NSLOTS * 16,), jnp.int32),
            pltpu.VMEM((NSLOTS * 16,), jnp.int32),
            pltpu.VMEM((ENT_CAP,), jnp.int32),
            pltpu.VMEM((NSLOTS,), jnp.int32),
            pltpu.VMEM((NSLOTS,), jnp.int32),
            pltpu.VMEM((NTILES * NSLOTS,), jnp.int32),
            pltpu.VMEM((NTILES * NSLOTS,), jnp.int32),
            pltpu.VMEM((CHUNK,), jnp.int32),
            pltpu.VMEM((GRP,), jnp.int32),
            pltpu.VMEM((GRP,), jnp.int32),
            pltpu.VMEM((GRP, FTBL), jnp.float32),
            pltpu.VMEM((SLOT_CELLS * PFN_OUT,), jnp.float32),
            pltpu.VMEM_SHARED((NTILES * ENT_ROW,), jnp.int32),
            pltpu.VMEM_SHARED((NTILES * NSLOTS,), jnp.int32),
            pltpu.VMEM_SHARED((NTILES * NSLOTS,), jnp.int32),
            pltpu.SemaphoreType.DMA,
        ],
    )(_scatter_body)
    bev = k(ids, feat.reshape(B * NPAD, FTBL))
    return bev.reshape(B, CELLS, PFN_OUT)


H2 = BEV_H // 2


def _conv1_body(par_ref, w_ref, b_ref, y1_ref):
    acc = jnp.zeros((H2 * H2, PFN_OUT), jnp.float32)
    for di in range(3):
        rp = di % 2
        rs = 1 if di == 2 else 0
        xa = par_ref[0, rp, rs:rs + H2, 0:H2, :].reshape(H2 * H2, 2 * PFN_OUT)
        acc = acc + jnp.dot(xa, w_ref[di * 2],
                            preferred_element_type=jnp.float32)
        xb = par_ref[0, rp, rs:rs + H2, 1:H2 + 1, :].reshape(
            H2 * H2, 2 * PFN_OUT)
        acc = acc + jnp.dot(xb, w_ref[di * 2 + 1],
                            preferred_element_type=jnp.float32)
    y1 = jax.nn.relu(acc + b_ref[...][None, :].astype(jnp.float32))
    y1_ref[0] = jnp.pad(y1.reshape(H2, H2, PFN_OUT),
                        ((1, 1), (1, 1), (0, 0)))


def _conv2_body(y1_ref, w_ref, b_ref, out_ref):
    acc = jnp.zeros((H2 * H2, BEV_CH), jnp.float32)
    for di in range(3):
        for dj in range(3):
            xk = y1_ref[0, di:di + H2, dj:dj + H2, :].reshape(
                H2 * H2, PFN_OUT)
            acc = acc + jnp.dot(xk, w_ref[di * 3 + dj],
                                preferred_element_type=jnp.float32)
    out_ref[0] = jax.nn.relu(acc + b_ref[...][None, :]).reshape(
        H2, H2, BEV_CH)


def _run_convs(par, w1p, b1, w2t, b2):
    y1 = pl.pallas_call(
        _conv1_body,
        grid=(B,),
        compiler_params=pltpu.CompilerParams(
            vmem_limit_bytes=100 * 1024 * 1024),
        in_specs=[
            pl.BlockSpec((1, 2, H2 + 1, H2 + 1, 2 * PFN_OUT),
                         lambda b: (b, 0, 0, 0, 0)),
            pl.BlockSpec((6, 2 * PFN_OUT, PFN_OUT), lambda b: (0, 0, 0)),
            pl.BlockSpec((PFN_OUT,), lambda b: (0,)),
        ],
        out_specs=pl.BlockSpec((1, H2 + 2, H2 + 2, PFN_OUT),
                               lambda b: (b, 0, 0, 0)),
        out_shape=jax.ShapeDtypeStruct((B, H2 + 2, H2 + 2, PFN_OUT),
                                       jnp.float32),
    )(par, w1p, b1)
    out = pl.pallas_call(
        _conv2_body,
        grid=(B,),
        compiler_params=pltpu.CompilerParams(
            vmem_limit_bytes=100 * 1024 * 1024),
        in_specs=[
            pl.BlockSpec((1, H2 + 2, H2 + 2, PFN_OUT),
                         lambda b: (b, 0, 0, 0)),
            pl.BlockSpec((9, PFN_OUT, BEV_CH), lambda b: (0, 0, 0)),
            pl.BlockSpec((BEV_CH,), lambda b: (0,)),
        ],
        out_specs=pl.BlockSpec((1, H2, H2, BEV_CH), lambda b: (b, 0, 0, 0)),
        out_shape=jax.ShapeDtypeStruct((B, H2, H2, BEV_CH), jnp.float32),
    )(y1, w2t, b2)
    return out


def kernel(points, W_pfn, b_pfn, conv1_w, conv1_b, conv2_w, conv2_b):
    pts = jnp.pad(points, ((0, 0), (0, NPAD - N), (0, 0)))
    ptsT = pts.transpose(0, 2, 1)
    w_pad = jnp.pad(W_pfn, ((0, 0), (0, FTBL - PFN_OUT)))
    b_pad = jnp.pad(b_pfn, (0, FTBL - PFN_OUT))
    feat, ids = _run_pfn(ptsT, w_pad, b_pad)

    bev = _run_scatter(ids, feat)

    x = bev.reshape(B, H2, 2, H2, 2, PFN_OUT)
    par = x.transpose(0, 2, 1, 3, 4, 5).reshape(B, 2, H2, H2, 2 * PFN_OUT)
    par = jnp.pad(par, ((0, 0), (0, 0), (0, 1), (0, 1), (0, 0)))
    par = par.astype(jnp.bfloat16)

    w1t = conv1_w.transpose(2, 3, 1, 0)
    wz = jnp.zeros((PFN_OUT, PFN_OUT), jnp.float32)
    w1p = jnp.stack([
        jnp.concatenate([w1t[0, 0], w1t[0, 1]], axis=0),
        jnp.concatenate([w1t[0, 2], wz], axis=0),
        jnp.concatenate([w1t[1, 0], w1t[1, 1]], axis=0),
        jnp.concatenate([w1t[1, 2], wz], axis=0),
        jnp.concatenate([w1t[2, 0], w1t[2, 1]], axis=0),
        jnp.concatenate([w1t[2, 2], wz], axis=0),
    ]).astype(jnp.bfloat16)
    w2t = conv2_w.transpose(2, 3, 1, 0).reshape(9, PFN_OUT, BEV_CH)
    out = _run_convs(par, w1p, conv1_b, w2t, conv2_b)
    return out.transpose(0, 3, 1, 2)


# speedup vs baseline: 2.9036x; 1.0143x over previous
import functools

import jax
import jax.numpy as jnp
from jax import lax
from jax.experimental import pallas as pl
from jax.experimental.pallas import tpu as pltpu
from jax.experimental.pallas import tpu_sc as plsc

B = 2
N = 100000
NPAD = 102400
BEV_H = 256
BEV_W = 256
CELLS = BEV_H * BEV_W
PFN_OUT = 64
BEV_CH = 128
FTBL = 128

NTILES = 16
PTS_PER_TILE = NPAD // NTILES
ASTEPS = PTS_PER_TILE // 16
NSLOTS = 64
SLOT_CELLS = CELLS // NSLOTS
NPASSES = NSLOTS // NTILES
ENT_CAP = PTS_PER_TILE + 8 * NSLOTS
ENT_ROW = ENT_CAP + 1024
CHUNK = 1024
GRP = 128
DUMMY_PIDX = NPAD - 1


def _pfn_body(ptsT_ref, w_ref, b_ref, feat_ref, ids_ref):
    ci = pl.program_id(1)
    x = ptsT_ref[0]
    f = jax.nn.relu(
        lax.dot_general(x[0:4, :], w_ref[...], (((0,), (0,)), ((), ())),
                        preferred_element_type=jnp.float32)
        + b_ref[...][None, :]
    )
    base = ci * f.shape[0]
    colv = base + lax.broadcasted_iota(jnp.int32, (f.shape[0], 1), 0)
    feat_ref[0] = jnp.where(colv < N, f, 0.0)
    col = base + lax.broadcasted_iota(jnp.int32, (1, f.shape[0]), 1)
    valid = col < N
    ix = jnp.clip((x[0:1, :] * BEV_W).astype(jnp.int32), 0, BEV_W - 1)
    iy = jnp.clip((x[1:2, :] * BEV_H).astype(jnp.int32), 0, BEV_H - 1)
    ids = iy * BEV_W + ix
    ids_ref[0] = jnp.where(valid, ids, CELLS - 1)


def _run_pfn(ptsT, W_pfn, b_pfn):
    CHN = 4096
    nch = NPAD // CHN
    feat, ids = pl.pallas_call(
        _pfn_body,
        grid=(B, nch),
        in_specs=[
            pl.BlockSpec((1, 5, CHN), lambda b, c: (b, 0, c)),
            pl.BlockSpec((4, FTBL), lambda b, c: (0, 0)),
            pl.BlockSpec((FTBL,), lambda b, c: (0,)),
        ],
        out_specs=[
            pl.BlockSpec((1, CHN, FTBL), lambda b, c: (b, c, 0)),
            pl.BlockSpec((1, 1, CHN), lambda b, c: (b, 0, c)),
        ],
        out_shape=[
            jax.ShapeDtypeStruct((B, NPAD, FTBL), jnp.float32),
            jax.ShapeDtypeStruct((B, 1, NPAD), jnp.int32),
        ],
    )(ptsT, W_pfn, b_pfn)
    return feat, ids


def _scatter_body(ids_hbm, feat_hbm, bev_hbm,
                  ids_v, counts2, offsets2, entries_v, starts_v, cnts_v,
                  startsm, cntsm, chunk_v, idx_v, cells_v, rows_v, slab,
                  entries_s, starts_s, cnts_s, sem):
    c = lax.axis_index("c")
    sid = lax.axis_index("s")
    lane = lax.iota(jnp.int32, 16)
    zeros16 = jnp.zeros((16,), jnp.int32)
    lane0 = lane < 1

    pltpu.sync_copy(ids_hbm.at[c, 0, pl.ds(sid * PTS_PER_TILE, PTS_PER_TILE)],
                    ids_v)

    def _zero(k, _):
        counts2[pl.ds(k * 16, 16)] = zeros16
        return 0
    lax.fori_loop(0, NSLOTS, _zero, 0)

    def _count(i, _):
        cv = ids_v[pl.ds(i * 16, 16)]
        slot16 = (lax.shift_right_logical(cv, 10) * 16) + lane
        cur = plsc.load_gather(counts2, [slot16])
        plsc.store_scatter(counts2, [slot16], cur + 1)
        return 0
    lax.fori_loop(0, ASTEPS, _count, 0)

    def _scan(r, carry):
        v = counts2[pl.ds(r * 16, 16)]
        s = plsc.cumsum(v)
        tot = jnp.sum(v)
        offsets2[pl.ds(r * 16, 16)] = (s - v) + carry
        r16 = jnp.broadcast_to(r, (16,))
        plsc.store_scatter(starts_v, [r16],
                           jnp.broadcast_to(carry, (16,)), mask=lane0)
        plsc.store_scatter(cnts_v, [r16],
                           jnp.broadcast_to(tot, (16,)), mask=lane0)
        return carry + ((tot + 7) & ~7)
    lax.fori_loop(0, NSLOTS, _scan, jnp.int32(0))

    def _place(i, _):
        cv = ids_v[pl.ds(i * 16, 16)]
        slot16 = (lax.shift_right_logical(cv, 10) * 16) + lane
        o = plsc.load_gather(offsets2, [slot16])
        plsc.store_scatter(offsets2, [slot16], o + 1)
        pidx = sid * PTS_PER_TILE + i * 16 + lane
        entry = ((cv & (SLOT_CELLS - 1)) << 17) | pidx
        plsc.store_scatter(entries_v, [o], entry)
        return 0
    lax.fori_loop(0, ASTEPS, _place, 0)

    pltpu.sync_copy(
        entries_v,
        entries_s.at[pl.ds(pl.multiple_of(sid * ENT_ROW, 8), ENT_CAP)])
    pltpu.sync_copy(
        starts_v,
        starts_s.at[pl.ds(pl.multiple_of(sid * NSLOTS, 8), NSLOTS)])
    pltpu.sync_copy(
        cnts_v,
        cnts_s.at[pl.ds(pl.multiple_of(sid * NSLOTS, 8), NSLOTS)])
    plsc.subcore_barrier()

    pltpu.sync_copy(starts_s, startsm)
    pltpu.sync_copy(cnts_s, cntsm)

    fbase = c * NPAD

    def _initidx(j, _):
        idx_v[pl.ds(j * 16, 16)] = N + j * 16 + lane
        cells_v[pl.ds(j * 16, 16)] = jnp.broadcast_to(SLOT_CELLS - 1, (16,))
        return 0
    lax.fori_loop(0, GRP // 16, _initidx, 0)

    for p in range(NPASSES):
        o = p * NTILES + sid
        o16 = jnp.broadcast_to(o, (16,))

        def _zslab(k, _):
            for u in range(4):
                slab[pl.ds(k * 64 + u * 16, 16)] = jnp.zeros((16,),
                                                             jnp.float32)
            return 0
        lax.fori_loop(0, SLOT_CELLS * PFN_OUT // 64, _zslab, 0)

        def _src(s, _):
            so16 = jnp.broadcast_to(s * NSLOTS, (16,)) + o16
            start = plsc.load_gather(startsm, [so16])[0]
            cnt = plsc.load_gather(cntsm, [so16])[0]

            def _chunk_cond(st):
                return st[0] < cnt

            def _chunk(st):
                done = st[0]
                pltpu.sync_copy(
                    entries_s.at[pl.ds(
                        pl.multiple_of(s * ENT_ROW + start + done, 8), CHUNK)],
                    chunk_v)
                navail = jnp.minimum(cnt - done, CHUNK)

                def _grp_cond(gt):
                    return gt[0] < navail

                def _grp(gt):
                    g0 = gt[0]
                    nblk = (jnp.minimum(navail - g0, GRP) + 15) // 16
                    def _prep(j, _):
                        e = chunk_v[pl.ds(g0 + j * 16, 16)]
                        pos = g0 + j * 16 + lane
                        ok = pos < navail
                        pidx = jnp.where(ok, e & (131072 - 1),
                                         N + j * 16 + lane)
                        cell = jnp.where(
                            ok, lax.shift_right_logical(e, 17), SLOT_CELLS - 1)
                        idx_v[pl.ds(j * 16, 16)] = pidx + fbase
                        cells_v[pl.ds(j * 16, 16)] = cell
                        return 0
                    lax.fori_loop(0, nblk, _prep, 0)
                    pltpu.async_copy(feat_hbm.at[idx_v], rows_v, sem).wait()

                    def _rmw(j, _):
                        cv16 = cells_v[pl.ds(j * 16, 16)]
                        for k in range(16):
                            cell = cv16[k]
                            base = cell * PFN_OUT
                            g = j * 16 + k
                            nq = PFN_OUT // 16
                            curs = [slab[pl.ds(base + q * 16, 16)]
                                    for q in range(nq)]
                            fvs = [rows_v[g, pl.ds(q * 16, 16)]
                                   for q in range(nq)]
                            for q in range(nq):
                                slab[pl.ds(base + q * 16, 16)] = \
                                    jnp.maximum(curs[q], fvs[q])
                        return 0
                    lax.fori_loop(0, nblk, _rmw, 0)
                    return (g0 + GRP,)
                lax.while_loop(_grp_cond, _grp, (jnp.int32(0),))
                return (done + CHUNK,)
            lax.while_loop(_chunk_cond, _chunk, (jnp.int32(0),))
            return 0
        lax.fori_loop(0, NTILES, _src, 0)

        pltpu.sync_copy(
            slab,
            bev_hbm.at[pl.ds(
                pl.multiple_of(
                    c * CELLS * PFN_OUT + o * SLOT_CELLS * PFN_OUT, 8),
                SLOT_CELLS * PFN_OUT)])


def _run_scatter(ids, feat):
    mesh = plsc.VectorSubcoreMesh(core_axis_name="c", subcore_axis_name="s")
    k = functools.partial(
        pl.kernel,
        mesh=mesh,
        compiler_params=pltpu.CompilerParams(needs_layout_passes=False),
        out_type=jax.ShapeDtypeStruct((B * CELLS * PFN_OUT,), jnp.float32),
        scratch_types=[
            pltpu.VMEM((PTS_PER_TILE,), jnp.int32),
            pltpu.VMEM((NSLOTS * 16,), jnp.int32),
            pltpu.VMEM((NSLOTS * 16,), jnp.int32),
            pltpu.VMEM((ENT_CAP,), jnp.int32),
            pltpu.VMEM((NSLOTS,), jnp.int32),
            pltpu.VMEM((NSLOTS,), jnp.int32),
            pltpu.VMEM((NTILES * NSLOTS,), jnp.int32),
            pltpu.VMEM((NTILES * NSLOTS,), jnp.int32),
            pltpu.VMEM((CHUNK,), jnp.int32),
            pltpu.VMEM((GRP,), jnp.int32),
            pltpu.VMEM((GRP,), jnp.int32),
            pltpu.VMEM((GRP, FTBL), jnp.float32),
            pltpu.VMEM((SLOT_CELLS * PFN_OUT,), jnp.float32),
            pltpu.VMEM_SHARED((NTILES * ENT_ROW,), jnp.int32),
            pltpu.VMEM_SHARED((NTILES * NSLOTS,), jnp.int32),
            pltpu.VMEM_SHARED((NTILES * NSLOTS,), jnp.int32),
            pltpu.SemaphoreType.DMA,
        ],
    )(_scatter_body)
    bev = k(ids, feat.reshape(B * NPAD, FTBL))
    return bev.reshape(B, CELLS, PFN_OUT)


H2 = BEV_H // 2


def _conv1_body(par_ref, w_ref, b_ref, y1_ref):
    acc = jnp.zeros((H2 * H2, PFN_OUT), jnp.float32)
    for di in range(3):
        rp = di % 2
        rs = 1 if di == 2 else 0
        xa = par_ref[0, rp, rs:rs + H2, 0:H2, :].reshape(H2 * H2, 2 * PFN_OUT)
        acc = acc + jnp.dot(xa, w_ref[di * 2],
                            preferred_element_type=jnp.float32)
        xb = par_ref[0, rp, rs:rs + H2, 1:H2 + 1, :].reshape(
            H2 * H2, 2 * PFN_OUT)
        acc = acc + jnp.dot(xb, w_ref[di * 2 + 1],
                            preferred_element_type=jnp.float32)
    y1 = jax.nn.relu(acc + b_ref[...][None, :].astype(jnp.float32))
    y1_ref[0] = jnp.pad(y1.reshape(H2, H2, PFN_OUT),
                        ((1, 1), (1, 1), (0, 0)))


def _conv2_body(y1_ref, w_ref, b_ref, out_ref):
    acc = jnp.zeros((H2 * H2, BEV_CH), jnp.float32)
    for di in range(3):
        for dj in range(3):
            xk = y1_ref[0, di:di + H2, dj:dj + H2, :].reshape(
                H2 * H2, PFN_OUT)
            acc = acc + jnp.dot(xk, w_ref[di * 3 + dj],
                                preferred_element_type=jnp.float32)
    out_ref[0] = jax.nn.relu(acc + b_ref[...][None, :]).reshape(
        H2, H2, BEV_CH)


def _run_convs(par, w1p, b1, w2t, b2):
    y1 = pl.pallas_call(
        _conv1_body,
        grid=(B,),
        compiler_params=pltpu.CompilerParams(
            vmem_limit_bytes=100 * 1024 * 1024),
        in_specs=[
            pl.BlockSpec((1, 2, H2 + 1, H2 + 1, 2 * PFN_OUT),
                         lambda b: (b, 0, 0, 0, 0)),
            pl.BlockSpec((6, 2 * PFN_OUT, PFN_OUT), lambda b: (0, 0, 0)),
            pl.BlockSpec((PFN_OUT,), lambda b: (0,)),
        ],
        out_specs=pl.BlockSpec((1, H2 + 2, H2 + 2, PFN_OUT),
                               lambda b: (b, 0, 0, 0)),
        out_shape=jax.ShapeDtypeStruct((B, H2 + 2, H2 + 2, PFN_OUT),
                                       jnp.float32),
    )(par, w1p, b1)
    out = pl.pallas_call(
        _conv2_body,
        grid=(B,),
        compiler_params=pltpu.CompilerParams(
            vmem_limit_bytes=100 * 1024 * 1024),
        in_specs=[
            pl.BlockSpec((1, H2 + 2, H2 + 2, PFN_OUT),
                         lambda b: (b, 0, 0, 0)),
            pl.BlockSpec((9, PFN_OUT, BEV_CH), lambda b: (0, 0, 0)),
            pl.BlockSpec((BEV_CH,), lambda b: (0,)),
        ],
        out_specs=pl.BlockSpec((1, H2, H2, BEV_CH), lambda b: (b, 0, 0, 0)),
        out_shape=jax.ShapeDtypeStruct((B, H2, H2, BEV_CH), jnp.float32),
    )(y1, w2t, b2)
    return out


def kernel(points, W_pfn, b_pfn, conv1_w, conv1_b, conv2_w, conv2_b):
    pts = jnp.pad(points, ((0, 0), (0, NPAD - N), (0, 0)))
    ptsT = pts.transpose(0, 2, 1)
    w_pad = jnp.pad(W_pfn, ((0, 0), (0, FTBL - PFN_OUT)))
    b_pad = jnp.pad(b_pfn, (0, FTBL - PFN_OUT))
    feat, ids = _run_pfn(ptsT, w_pad, b_pad)

    bev = _run_scatter(ids, feat)

    x = bev.reshape(B, H2, 2, H2, 2, PFN_OUT)
    par = x.transpose(0, 2, 1, 3, 4, 5).reshape(B, 2, H2, H2, 2 * PFN_OUT)
    par = jnp.pad(par, ((0, 0), (0, 0), (0, 1), (0, 1), (0, 0)))
    par = par.astype(jnp.bfloat16)

    w1t = conv1_w.transpose(2, 3, 1, 0)
    wz = jnp.zeros((PFN_OUT, PFN_OUT), jnp.float32)
    w1p = jnp.stack([
        jnp.concatenate([w1t[0, 0], w1t[0, 1]], axis=0),
        jnp.concatenate([w1t[0, 2], wz], axis=0),
        jnp.concatenate([w1t[1, 0], w1t[1, 1]], axis=0),
        jnp.concatenate([w1t[1, 2], wz], axis=0),
        jnp.concatenate([w1t[2, 0], w1t[2, 1]], axis=0),
        jnp.concatenate([w1t[2, 2], wz], axis=0),
    ]).astype(jnp.bfloat16)
    w2t = conv2_w.transpose(2, 3, 1, 0).reshape(9, PFN_OUT, BEV_CH)
    out = _run_convs(par, w1p, conv1_b, w2t, conv2_b)
    return out.transpose(0, 3, 1, 2)
